# baseline (speedup 1.0000x reference)
import functools

import jax
import jax.numpy as jnp
from jax import lax
from jax.experimental import pallas as pl
from jax.experimental.pallas import tpu as pltpu
from jax.experimental.pallas import tpu_sc as plsc

N_CORES = 2
N_SUBCORES = 16
LANES = 16
N_WORKERS = N_CORES * N_SUBCORES


def _sc_body(bpw, rank, hidx_hbm, ridx_hbm, tidx_hbm, bh_hbm, bt_hbm,
             ent_hbm, rel_hbm, pred_hbm, hout_hbm, rout_hbm, tout_hbm,
             hidx_v, ridx_v, tidx_v, hrows_v, rrows_v, trows_v,
             bh_v, bt_v, pred_v, pb_v,
             sem_h, sem_r, sem_t, sem_bh, sem_bt, sem_oh, sem_or, sem_ot):
  wid = lax.axis_index("s") * N_CORES + lax.axis_index("c")
  base = wid * bpw

  pltpu.sync_copy(hidx_hbm.at[pl.ds(base, bpw)], hidx_v)
  pltpu.sync_copy(ridx_hbm.at[pl.ds(base, bpw)], ridx_v)
  pltpu.sync_copy(tidx_hbm.at[pl.ds(base, bpw)], tidx_v)

  cp_h = pltpu.make_async_copy(ent_hbm.at[hidx_v], hrows_v, sem_h)
  cp_r = pltpu.make_async_copy(rel_hbm.at[ridx_v], rrows_v, sem_r)
  cp_t = pltpu.make_async_copy(ent_hbm.at[tidx_v], trows_v, sem_t)
  cp_bh = pltpu.make_async_copy(bh_hbm.at[hidx_v], bh_v, sem_bh)
  cp_bt = pltpu.make_async_copy(bt_hbm.at[tidx_v], bt_v, sem_bt)
  cp_h.start()
  cp_r.start()
  cp_t.start()
  cp_bh.start()
  cp_bt.start()
  cp_h.wait()
  cp_r.wait()
  cp_t.wait()

  oc_h = pltpu.make_async_copy(hrows_v, hout_hbm.at[pl.ds(base, bpw)], sem_oh)
  oc_r = pltpu.make_async_copy(rrows_v, rout_hbm.at[pl.ds(base, bpw)], sem_or)
  oc_t = pltpu.make_async_copy(trows_v, tout_hbm.at[pl.ds(base, bpw)], sem_ot)
  oc_h.start()
  oc_r.start()
  oc_t.start()
  cp_bh.wait()
  cp_bt.wait()

  n_chunks = rank // LANES
  lane = lax.iota(jnp.int32, LANES)

  def group_body(gi, _):
    g0 = gi * LANES
    for j in range(LANES):
      row = g0 + j
      p = jnp.zeros((LANES,), jnp.float32)
      for k in range(n_chunks):
        sl = pl.ds(k * LANES, LANES)
        p = p + hrows_v[row, sl] * rrows_v[row, sl] * trows_v[row, sl]
      pb_v[pl.ds(j * LANES, LANES)] = p
    acc = jnp.zeros((LANES,), jnp.float32)
    for l in range(LANES):
      acc = acc + plsc.load_gather(pb_v, [lane * LANES + l])
    gsl = pl.ds(g0, LANES)
    pred_v[gsl] = acc + bh_v[gsl] + bt_v[gsl]
    return 0

  lax.fori_loop(0, bpw // LANES, group_body, 0)

  pltpu.sync_copy(pred_v, pred_hbm.at[pl.ds(base, bpw)])
  oc_h.wait()
  oc_r.wait()
  oc_t.wait()


def kernel(queries, tails, entity_w, rel_w, bh_w, bt_w):
  b = queries.shape[0]
  rank = entity_w.shape[1]
  bpw = b // N_WORKERS

  head_idx = queries[:, 0]
  rel_idx = queries[:, 1]
  tail_idx = tails[:, 0]
  bh_flat = bh_w[:, 0]
  bt_flat = bt_w[:, 0]

  mesh = plsc.VectorSubcoreMesh(core_axis_name="c", subcore_axis_name="s")
  f32 = jnp.float32
  run = pl.kernel(
      functools.partial(_sc_body, bpw, rank),
      out_type=(
          jax.ShapeDtypeStruct((b,), f32),
          jax.ShapeDtypeStruct((b, rank), f32),
          jax.ShapeDtypeStruct((b, rank), f32),
          jax.ShapeDtypeStruct((b, rank), f32),
      ),
      mesh=mesh,
      compiler_params=pltpu.CompilerParams(
          needs_layout_passes=False, use_tc_tiling_on_sc=False),
      scratch_types=[
          pltpu.VMEM((bpw,), jnp.int32),
          pltpu.VMEM((bpw,), jnp.int32),
          pltpu.VMEM((bpw,), jnp.int32),
          pltpu.VMEM((bpw, rank), f32),
          pltpu.VMEM((bpw, rank), f32),
          pltpu.VMEM((bpw, rank), f32),
          pltpu.VMEM((bpw,), f32),
          pltpu.VMEM((bpw,), f32),
          pltpu.VMEM((bpw,), f32),
          pltpu.VMEM((LANES * LANES,), f32),
      ] + [pltpu.SemaphoreType.DMA] * 8,
  )
  pred, head_e, rel_e, rhs_e = run(
      head_idx, rel_idx, tail_idx, bh_flat, bt_flat, entity_w, rel_w)

  predictions = pred.reshape(b, 1, 1)
  return (predictions,
          head_e.reshape(b, 1, rank),
          rel_e.reshape(b, 1, rank),
          rhs_e.reshape(b, 1, rank))
